# baseline (speedup 1.0000x reference)
import functools

import jax
import jax.numpy as jnp
from jax import lax
from jax.experimental import pallas as pl
from jax.experimental.pallas import tpu as pltpu
from jax.experimental.pallas import tpu_sc as plsc

NC = 2
NS = 16
LANES = 16
NW = NC * NS
CHUNK = 64
NBUF = 4
ROWSEG = 128


def _cdiv(a, b):
  return (a + b - 1) // b


def _zero_rows(rows_v, nrows, d):
  def zr(r, _):
    for j in range(d // LANES):
      rows_v[r, pl.ds(j * LANES, LANES)] = jnp.zeros((LANES,), jnp.float32)
    return 0
  lax.fori_loop(0, nrows, zr, 0)


def _scale_rows(rows_b, load_w16, d):
  def scale_g(g, _):
    w16 = load_w16(g)
    e0 = g * LANES
    for l in range(LANES):
      we = w16[l]
      for j in range(d // LANES):
        sl = pl.ds(j * LANES, LANES)
        rows_b[e0 + l, sl] = rows_b[e0 + l, sl] * we
    return 0
  lax.fori_loop(0, CHUNK // LANES, scale_g, 0)


def _sc_pipeline(nch, wid, load_edata, start_gather, wait_gather,
                 process, start_scatter, wait_scatter):
  assert nch % 4 == 0 and nch >= 8

  def prefetch(k, b):
    load_edata(k, b)
    start_gather(k, b)

  def consume(k, b):
    wait_gather(k, b)
    process(k, b)
    start_scatter(k, b)

  prefetch(0, 0)
  prefetch(1, 1)
  prefetch(2, 2)
  consume(0, 0)
  prefetch(3, 3)
  consume(1, 1)

  def quad(t, _):
    i0 = 4 * t + 2
    for j in range(4):
      b_p = (2 + j + 2) % 4
      b_c = (2 + j) % 4
      i = i0 + j
      wait_scatter(b_p)
      prefetch(i + 2, b_p)
      consume(i, b_c)
    return 0
  lax.fori_loop(0, (nch - 4) // 4, quad, 0)

  consume(nch - 2, (nch - 2) % 4)
  consume(nch - 1, (nch - 1) % 4)
  for b in range(4):
    wait_scatter(b)


def _make_sc1(n_nodes, nt_rows, n_pad, nch, d):
  mesh = plsc.VectorSubcoreMesh(
      core_axis_name="c", subcore_axis_name="s", num_cores=NC, num_subcores=NS)
  rows_pt = n_pad // NS
  seg_pt = rows_pt // ROWSEG

  @functools.partial(
      pl.kernel,
      out_type=(
          jax.ShapeDtypeStruct((NC, n_pad, d), jnp.float32),
          jax.ShapeDtypeStruct((NC, n_pad), jnp.float32),
          jax.ShapeDtypeStruct((NW, nch, CHUNK), jnp.float32),
      ),
      mesh=mesh,
      compiler_params=pltpu.CompilerParams(needs_layout_passes=False),
      scratch_types=(
          [pltpu.VMEM((4, CHUNK), jnp.int32) for _ in range(NBUF)] +
          [pltpu.VMEM((CHUNK,), jnp.float32) for _ in range(NBUF)] +
          [pltpu.VMEM((CHUNK, d), jnp.float32) for _ in range(NBUF)] +
          [
              pltpu.VMEM((nt_rows, ROWSEG), jnp.float32),
              pltpu.VMEM((rows_pt,), jnp.float32),
              pltpu.VMEM_SHARED((n_pad, d), jnp.float32),
              pltpu.VMEM_SHARED((n_pad,), jnp.float32),
          ] +
          [pltpu.SemaphoreType.DMA for _ in range(2 * NBUF)]
      ),
  )
  def sc1(x_hbm, edata_hbm, nt_hbm, agg_hbm, deg_hbm, w_hbm, *scr):
    eb = scr[0:NBUF]
    wc = scr[NBUF:2 * NBUF]
    rows = scr[2 * NBUF:3 * NBUF]
    nt_v, zdeg_v, agg_s, deg_s = scr[3 * NBUF:3 * NBUF + 4]
    gsem = scr[3 * NBUF + 4:3 * NBUF + 4 + NBUF]
    ssem = scr[3 * NBUF + 4 + NBUF:]

    cid = lax.axis_index("c")
    sid = lax.axis_index("s")
    wid = cid * NS + sid

    pltpu.sync_copy(nt_hbm, nt_v)

    _zero_rows(rows[0], CHUNK, d)
    _zero_rows(rows[1], CHUNK, d)

    def zd(i, _):
      zdeg_v[pl.ds(i * LANES, LANES)] = jnp.zeros((LANES,), jnp.float32)
      return 0
    lax.fori_loop(0, rows_pt // LANES, zd, 0)

    base = sid * rows_pt
    for s in range(seg_pt):
      pltpu.sync_copy(rows[s % 2], agg_s.at[pl.ds(base + s * ROWSEG, CHUNK)])
      pltpu.sync_copy(rows[s % 2],
                      agg_s.at[pl.ds(base + s * ROWSEG + CHUNK, CHUNK)])
    pltpu.sync_copy(zdeg_v, deg_s.at[pl.ds(base, rows_pt)])
    plsc.subcore_barrier()

    def load_edata(k, b):
      pltpu.sync_copy(edata_hbm.at[wid, k], eb[b])

    def start_gather(k, b):
      pltpu.async_copy(x_hbm.at[eb[b].at[0]], rows[b], gsem[b])

    def wait_gather(k, b):
      pltpu.make_async_copy(x_hbm.at[eb[b].at[0]], rows[b], gsem[b]).wait()

    def start_scatter(k, b):
      pltpu.async_copy(rows[b], agg_s.at[eb[b].at[1]], ssem[b], add=True)

    def wait_scatter(b):
      pltpu.make_async_copy(rows[b], agg_s.at[eb[b].at[1]], ssem[b]).wait()

    def process(k, b):
      def sub(kk, _):
        sl = pl.ds(kk * LANES, LANES)
        dd = eb[b][1, sl]
        ntg = plsc.load_gather(
            nt_v, [lax.shift_right_logical(dd, 7), lax.bitwise_and(dd, 127)])
        dt = ntg - plsc.bitcast(eb[b][2, sl], jnp.float32)
        wcb = wc[b]
        wcb[sl] = jnp.exp(-jnp.abs(dt)) * plsc.bitcast(eb[b][3, sl], jnp.float32)
        return 0
      lax.fori_loop(0, CHUNK // LANES, sub, 0)

      pltpu.sync_copy(wc[b], w_hbm.at[wid, k])
      pltpu.sync_copy(wc[b], deg_s.at[eb[b].at[1]], add=True)

      def w16_of(g):
        return wc[b][pl.ds(g * LANES, LANES)]
      _scale_rows(rows[b], w16_of, d)

    _sc_pipeline(nch, wid, load_edata, start_gather, wait_gather,
                 process, start_scatter, wait_scatter)

    plsc.subcore_barrier()

    for s in range(seg_pt):
      sl = pl.ds(base + s * ROWSEG, ROWSEG)
      pltpu.sync_copy(agg_s.at[sl], agg_hbm.at[cid, sl])
    pltpu.sync_copy(deg_s.at[pl.ds(base, rows_pt)],
                    deg_hbm.at[cid, pl.ds(base, rows_pt)])

  return sc1


def _make_sc2(n_pad, nch, d):
  mesh = plsc.VectorSubcoreMesh(
      core_axis_name="c", subcore_axis_name="s", num_cores=NC, num_subcores=NS)
  rows_pt = n_pad // NS
  seg_pt = rows_pt // ROWSEG

  @functools.partial(
      pl.kernel,
      out_type=jax.ShapeDtypeStruct((NC, n_pad, d), jnp.float32),
      mesh=mesh,
      compiler_params=pltpu.CompilerParams(needs_layout_passes=False),
      scratch_types=(
          [pltpu.VMEM((3, CHUNK), jnp.int32) for _ in range(NBUF)] +
          [pltpu.VMEM((CHUNK, d), jnp.float32) for _ in range(NBUF)] +
          [pltpu.VMEM_SHARED((n_pad, d), jnp.float32)] +
          [pltpu.SemaphoreType.DMA for _ in range(2 * NBUF)]
      ),
  )
  def sc2(h_hbm, edata_hbm, agg_hbm, *scr):
    eb = scr[0:NBUF]
    rows = scr[NBUF:2 * NBUF]
    agg_s = scr[2 * NBUF]
    gsem = scr[2 * NBUF + 1:2 * NBUF + 1 + NBUF]
    ssem = scr[2 * NBUF + 1 + NBUF:]

    cid = lax.axis_index("c")
    sid = lax.axis_index("s")
    wid = cid * NS + sid

    _zero_rows(rows[0], CHUNK, d)
    _zero_rows(rows[1], CHUNK, d)
    base = sid * rows_pt
    for s in range(seg_pt):
      pltpu.sync_copy(rows[s % 2], agg_s.at[pl.ds(base + s * ROWSEG, CHUNK)])
      pltpu.sync_copy(rows[s % 2],
                      agg_s.at[pl.ds(base + s * ROWSEG + CHUNK, CHUNK)])
    plsc.subcore_barrier()

    def load_edata(k, b):
      pltpu.sync_copy(edata_hbm.at[wid, k], eb[b])

    def start_gather(k, b):
      pltpu.async_copy(h_hbm.at[eb[b].at[0]], rows[b], gsem[b])

    def wait_gather(k, b):
      pltpu.make_async_copy(h_hbm.at[eb[b].at[0]], rows[b], gsem[b]).wait()

    def start_scatter(k, b):
      pltpu.async_copy(rows[b], agg_s.at[eb[b].at[1]], ssem[b], add=True)

    def wait_scatter(b):
      pltpu.make_async_copy(rows[b], agg_s.at[eb[b].at[1]], ssem[b]).wait()

    def process(k, b):
      def w16_of(g):
        return plsc.bitcast(eb[b][2, pl.ds(g * LANES, LANES)], jnp.float32)
      _scale_rows(rows[b], w16_of, d)

    _sc_pipeline(nch, wid, load_edata, start_gather, wait_gather,
                 process, start_scatter, wait_scatter)

    plsc.subcore_barrier()
    for s in range(seg_pt):
      sl = pl.ds(base + s * ROWSEG, ROWSEG)
      pltpu.sync_copy(agg_s.at[sl], agg_hbm.at[cid, sl])

  return sc2


def _tc1_body(agg_ref, degt_ref, w_ref, b_ref, out_ref):
  a = agg_ref[0] + agg_ref[1]
  dsum = degt_ref[:, 0:1] + degt_ref[:, 1:2]
  m = a / jnp.maximum(dsum, 1e-6)
  h = jnp.dot(m, w_ref[...], preferred_element_type=jnp.float32) + b_ref[...]
  out_ref[...] = jnp.where(h >= 0, h, 0.01 * h)


def _make_tc1(n_pad, d, blk):
  grid = (n_pad // blk,)
  return pl.pallas_call(
      _tc1_body,
      grid=grid,
      in_specs=[
          pl.BlockSpec((NC, blk, d), lambda i: (0, i, 0)),
          pl.BlockSpec((blk, NC), lambda i: (i, 0)),
          pl.BlockSpec((d, d), lambda i: (0, 0)),
          pl.BlockSpec((1, d), lambda i: (0, 0)),
      ],
      out_specs=pl.BlockSpec((blk, d), lambda i: (i, 0)),
      out_shape=jax.ShapeDtypeStruct((n_pad, d), jnp.float32),
  )


def _make_tc2(n_nodes, n_pad, d):
  def body(agg_ref, degt_ref, w2_ref, b2_ref, gamma_ref, beta_ref,
           wf1_ref, bf1_ref, wf2_ref, bf2_ref, out_ref):
    a = agg_ref[0] + agg_ref[1]
    dsum = degt_ref[:, 0:1] + degt_ref[:, 1:2]
    m = a / jnp.maximum(dsum, 1e-6)
    h = jnp.dot(m, w2_ref[...], preferred_element_type=jnp.float32) + b2_ref[...]
    h = jnp.where(h >= 0, h, 0.01 * h)
    rid = lax.broadcasted_iota(jnp.int32, (n_pad, d), 0)
    msk = rid < n_nodes
    hm = jnp.where(msk, h, 0.0)
    inv_n = 1.0 / n_nodes
    mu = jnp.sum(hm, axis=0, keepdims=True) * inv_n
    ex2 = jnp.sum(hm * hm, axis=0, keepdims=True) * inv_n
    var = ex2 - mu * mu
    hb = (h - mu) / jnp.sqrt(var + 1e-5) * gamma_ref[...] + beta_ref[...]
    hb = jnp.where(hb >= 0, hb, 0.01 * hb)
    h4 = jnp.dot(hb, wf1_ref[...], preferred_element_type=jnp.float32) + bf1_ref[...]
    h4 = jnp.where(h4 >= 0, h4, 0.01 * h4)
    out_ref[...] = (
        jnp.dot(h4, wf2_ref[...], preferred_element_type=jnp.float32)
        + bf2_ref[...])

  return pl.pallas_call(
      body,
      out_shape=jax.ShapeDtypeStruct((n_pad, d), jnp.float32),
  )


def kernel(x, edge_index, edge_time, node_time, edge_weight,
           W1, b1, W2, b2, gamma, beta, Wf1, bf1, Wf2, bf2):
  n_nodes, d = x.shape
  e = edge_index.shape[1]
  out_dim = Wf2.shape[1]

  n_pad = _cdiv(n_nodes, NS * ROWSEG) * NS * ROWSEG
  nch = 4 * _cdiv(e, NW * CHUNK * 4)
  e_pad = nch * NW * CHUNK
  pad = e_pad - e

  src = edge_index[0].astype(jnp.int32)
  dst = edge_index[1].astype(jnp.int32)
  zi = jnp.zeros((pad,), jnp.int32)
  zf = jnp.zeros((pad,), jnp.float32)
  src3 = jnp.concatenate([src, zi]).reshape(NW, nch, CHUNK)
  dst3 = jnp.concatenate([dst, zi]).reshape(NW, nch, CHUNK)
  et3 = jnp.concatenate([edge_time.astype(jnp.float32), zf]).reshape(NW, nch, CHUNK)
  ew3 = jnp.concatenate([edge_weight.astype(jnp.float32), zf]).reshape(NW, nch, CHUNK)
  edata1 = jnp.stack([
      src3, dst3,
      lax.bitcast_convert_type(et3, jnp.int32),
      lax.bitcast_convert_type(ew3, jnp.int32),
  ], axis=2)

  nt_rows = _cdiv(n_nodes, ROWSEG)
  nt2 = jnp.pad(node_time.astype(jnp.float32),
                (0, nt_rows * ROWSEG - n_nodes)).reshape(nt_rows, ROWSEG)
  sc1 = _make_sc1(n_nodes, nt_rows, n_pad, nch, d)
  sc2 = _make_sc2(n_pad, nch, d)
  tc1 = _make_tc1(n_pad, d, 1024)
  tc2 = _make_tc2(n_nodes, n_pad, d)

  agg1, deg, w3 = sc1(x, edata1, nt2)
  degt = deg.T

  h1 = tc1(agg1, degt, W1, b1.reshape(1, d))

  edata2 = jnp.stack(
      [src3, dst3, lax.bitcast_convert_type(w3, jnp.int32)], axis=2)
  agg2 = sc2(h1, edata2)

  wf1p = jnp.pad(Wf1, ((0, 0), (0, d - Wf1.shape[1])))
  bf1p = jnp.pad(bf1, (0, d - bf1.shape[0])).reshape(1, d)
  wf2p = jnp.pad(Wf2, ((0, d - Wf2.shape[0]), (0, d - Wf2.shape[1])))
  bf2p = jnp.pad(bf2, (0, d - bf2.shape[0])).reshape(1, d)

  out = tc2(agg2, degt, W2, b2.reshape(1, d),
            gamma.reshape(1, d), beta.reshape(1, d),
            wf1p, bf1p, wf2p, bf2p)
  return out[:n_nodes, :out_dim]


# speedup vs baseline: 1.8950x; 1.8950x over previous
import functools

import jax
import jax.numpy as jnp
from jax import lax
from jax.experimental import pallas as pl
from jax.experimental.pallas import tpu as pltpu
from jax.experimental.pallas import tpu_sc as plsc

NC = 2
NS = 16
LANES = 16
NW = NC * NS
CHUNK = 128
CORE0_SHARE = 0.7


def _cdiv(a, b):
  return (a + b - 1) // b


def _chunk_split(e):
  total = 2 * _cdiv(e, NS * CHUNK * 2)
  nch0 = min(total - 2, max(2, 2 * int(round(total * CORE0_SHARE / 2))))
  return nch0, total - nch0


def _zero_rows(rows_v, d):
  def zr(r, _):
    for j in range(d // LANES):
      rows_v[r, pl.ds(j * LANES, LANES)] = jnp.zeros((LANES,), jnp.float32)
    return 0
  lax.fori_loop(0, CHUNK, zr, 0)


def _scale_rows(rows_b, load_w16, d):
  def scale_g(g, _):
    w16 = load_w16(g)
    e0 = g * LANES
    for l in range(LANES):
      we = w16[l]
      for j in range(d // LANES):
        sl = pl.ds(j * LANES, LANES)
        rows_b[e0 + l, sl] = rows_b[e0 + l, sl] * we
    return 0
  lax.fori_loop(0, CHUNK // LANES, scale_g, 0)


def _sc_pipeline(nch_c, load_edata, start_gather, process, wait_scatter):
  load_edata(0, 0)
  start_gather(0)
  load_edata(1, 1)
  start_gather(1)
  process(0, 0)

  def pair(t, _):
    k = 2 * t + 1
    wait_scatter(0)
    load_edata(k + 1, 0)
    start_gather(0)
    process(k, 1)
    wait_scatter(1)
    load_edata(k + 2, 1)
    start_gather(1)
    process(k + 1, 0)
    return 0
  lax.fori_loop(0, nch_c // 2 - 1, pair, 0)

  wait_scatter(0)
  process(nch_c - 1, 1)
  wait_scatter(1)


def _make_sc1(n_nodes, nt_rows, n_pad, nch0, nch1, d):
  mesh = plsc.VectorSubcoreMesh(
      core_axis_name="c", subcore_axis_name="s", num_cores=NC, num_subcores=NS)
  rows_pt = n_pad // NS
  seg_pt = rows_pt // CHUNK
  nch_max = max(nch0, nch1)

  @functools.partial(
      pl.kernel,
      out_type=(
          jax.ShapeDtypeStruct((NC, n_pad, d), jnp.float32),
          jax.ShapeDtypeStruct((NC, n_pad), jnp.float32),
          jax.ShapeDtypeStruct((NC, NS, nch_max, CHUNK), jnp.float32),
      ),
      mesh=mesh,
      compiler_params=pltpu.CompilerParams(needs_layout_passes=False),
      scratch_types=[
          pltpu.VMEM((4, CHUNK), jnp.int32),
          pltpu.VMEM((4, CHUNK), jnp.int32),
          pltpu.VMEM((CHUNK,), jnp.float32),
          pltpu.VMEM((CHUNK,), jnp.float32),
          pltpu.VMEM((CHUNK, d), jnp.float32),
          pltpu.VMEM((CHUNK, d), jnp.float32),
          pltpu.VMEM((nt_rows, CHUNK), jnp.float32),
          pltpu.VMEM((rows_pt,), jnp.float32),
          pltpu.VMEM_SHARED((n_pad, d), jnp.float32),
          pltpu.VMEM_SHARED((n_pad,), jnp.float32),
          pltpu.SemaphoreType.DMA,
          pltpu.SemaphoreType.DMA,
          pltpu.SemaphoreType.DMA,
          pltpu.SemaphoreType.DMA,
      ],
  )
  def sc1(x_hbm, edata_hbm, nt_hbm,
          agg_hbm, deg_hbm, w_hbm,
          eb0, eb1, w0, w1, rows0, rows1, nt_v, zdeg_v,
          agg_s, deg_s, gsem0, gsem1, ssem0, ssem1):
    cid = lax.axis_index("c")
    sid = lax.axis_index("s")
    nch_c = jnp.where(cid == 0, nch0, nch1)
    eb = (eb0, eb1)
    wc = (w0, w1)
    rows = (rows0, rows1)
    gsem = (gsem0, gsem1)
    ssem = (ssem0, ssem1)

    pltpu.sync_copy(nt_hbm, nt_v)

    _zero_rows(rows0, d)

    def zd(i, _):
      zdeg_v[pl.ds(i * LANES, LANES)] = jnp.zeros((LANES,), jnp.float32)
      return 0
    lax.fori_loop(0, rows_pt // LANES, zd, 0)

    base = sid * rows_pt
    for s in range(seg_pt):
      pltpu.sync_copy(rows0, agg_s.at[pl.ds(base + s * CHUNK, CHUNK)])
    pltpu.sync_copy(zdeg_v, deg_s.at[pl.ds(base, rows_pt)])
    plsc.subcore_barrier()

    def load_edata(k, b):
      pltpu.sync_copy(edata_hbm.at[cid, sid, k], eb[b])

    def start_gather(b):
      pltpu.async_copy(x_hbm.at[eb[b].at[0]], rows[b], gsem[b])

    def wait_scatter(b):
      pltpu.make_async_copy(rows[b], agg_s.at[eb[b].at[1]], ssem[b]).wait()

    def process(k, b):
      pltpu.make_async_copy(x_hbm.at[eb[b].at[0]], rows[b], gsem[b]).wait()

      def sub(kk, _):
        sl = pl.ds(kk * LANES, LANES)
        dd = eb[b][1, sl]
        ntg = plsc.load_gather(
            nt_v, [lax.shift_right_logical(dd, 7), lax.bitwise_and(dd, 127)])
        dt = ntg - plsc.bitcast(eb[b][2, sl], jnp.float32)
        wcb = wc[b]
        wcb[sl] = jnp.exp(-jnp.abs(dt)) * plsc.bitcast(eb[b][3, sl], jnp.float32)
        return 0
      lax.fori_loop(0, CHUNK // LANES, sub, 0)

      pltpu.sync_copy(wc[b], w_hbm.at[cid, sid, k])
      pltpu.sync_copy(wc[b], deg_s.at[eb[b].at[1]], add=True)

      def w16_of(g):
        return wc[b][pl.ds(g * LANES, LANES)]
      _scale_rows(rows[b], w16_of, d)
      pltpu.async_copy(rows[b], agg_s.at[eb[b].at[1]], ssem[b], add=True)

    _sc_pipeline(nch_c, load_edata,
                 lambda b: start_gather(b), process, wait_scatter)

    plsc.subcore_barrier()

    for s in range(seg_pt):
      sl = pl.ds(base + s * CHUNK, CHUNK)
      pltpu.sync_copy(agg_s.at[sl], agg_hbm.at[cid, sl])
    pltpu.sync_copy(deg_s.at[pl.ds(base, rows_pt)],
                    deg_hbm.at[cid, pl.ds(base, rows_pt)])

  return sc1


def _make_sc2(n_pad, nch0, nch1, d):
  mesh = plsc.VectorSubcoreMesh(
      core_axis_name="c", subcore_axis_name="s", num_cores=NC, num_subcores=NS)
  rows_pt = n_pad // NS
  seg_pt = rows_pt // CHUNK

  @functools.partial(
      pl.kernel,
      out_type=jax.ShapeDtypeStruct((NC, n_pad, d), jnp.float32),
      mesh=mesh,
      compiler_params=pltpu.CompilerParams(needs_layout_passes=False),
      scratch_types=[
          pltpu.VMEM((3, CHUNK), jnp.int32),
          pltpu.VMEM((3, CHUNK), jnp.int32),
          pltpu.VMEM((CHUNK, d), jnp.float32),
          pltpu.VMEM((CHUNK, d), jnp.float32),
          pltpu.VMEM_SHARED((n_pad, d), jnp.float32),
          pltpu.SemaphoreType.DMA,
          pltpu.SemaphoreType.DMA,
          pltpu.SemaphoreType.DMA,
          pltpu.SemaphoreType.DMA,
      ],
  )
  def sc2(h_hbm, edata_hbm, agg_hbm,
          eb0, eb1, rows0, rows1, agg_s, gsem0, gsem1, ssem0, ssem1):
    cid = lax.axis_index("c")
    sid = lax.axis_index("s")
    nch_c = jnp.where(cid == 0, nch0, nch1)
    eb = (eb0, eb1)
    rows = (rows0, rows1)
    gsem = (gsem0, gsem1)
    ssem = (ssem0, ssem1)

    _zero_rows(rows0, d)
    base = sid * rows_pt
    for s in range(seg_pt):
      pltpu.sync_copy(rows0, agg_s.at[pl.ds(base + s * CHUNK, CHUNK)])
    plsc.subcore_barrier()

    def load_edata(k, b):
      pltpu.sync_copy(edata_hbm.at[cid, sid, k], eb[b])

    def start_gather(b):
      pltpu.async_copy(h_hbm.at[eb[b].at[0]], rows[b], gsem[b])

    def wait_scatter(b):
      pltpu.make_async_copy(rows[b], agg_s.at[eb[b].at[1]], ssem[b]).wait()

    def process(k, b):
      pltpu.make_async_copy(h_hbm.at[eb[b].at[0]], rows[b], gsem[b]).wait()

      def w16_of(g):
        return plsc.bitcast(eb[b][2, pl.ds(g * LANES, LANES)], jnp.float32)
      _scale_rows(rows[b], w16_of, d)
      pltpu.async_copy(rows[b], agg_s.at[eb[b].at[1]], ssem[b], add=True)

    _sc_pipeline(nch_c, load_edata,
                 lambda b: start_gather(b), process, wait_scatter)

    plsc.subcore_barrier()
    for s in range(seg_pt):
      sl = pl.ds(base + s * CHUNK, CHUNK)
      pltpu.sync_copy(agg_s.at[sl], agg_hbm.at[cid, sl])

  return sc2


def _tc1_body(agg_ref, degt_ref, w_ref, b_ref, out_ref):
  a = agg_ref[0] + agg_ref[1]
  dsum = degt_ref[:, 0:1] + degt_ref[:, 1:2]
  m = a / jnp.maximum(dsum, 1e-6)
  h = jnp.dot(m, w_ref[...], preferred_element_type=jnp.float32) + b_ref[...]
  out_ref[...] = jnp.where(h >= 0, h, 0.01 * h)


def _make_tc1(n_pad, d, blk):
  grid = (n_pad // blk,)
  return pl.pallas_call(
      _tc1_body,
      grid=grid,
      in_specs=[
          pl.BlockSpec((NC, blk, d), lambda i: (0, i, 0)),
          pl.BlockSpec((blk, NC), lambda i: (i, 0)),
          pl.BlockSpec((d, d), lambda i: (0, 0)),
          pl.BlockSpec((1, d), lambda i: (0, 0)),
      ],
      out_specs=pl.BlockSpec((blk, d), lambda i: (i, 0)),
      out_shape=jax.ShapeDtypeStruct((n_pad, d), jnp.float32),
  )


def _make_tc2(n_nodes, n_pad, d):
  def body(agg_ref, degt_ref, w2_ref, b2_ref, gamma_ref, beta_ref,
           wf1_ref, bf1_ref, wf2_ref, bf2_ref, out_ref):
    a = agg_ref[0] + agg_ref[1]
    dsum = degt_ref[:, 0:1] + degt_ref[:, 1:2]
    m = a / jnp.maximum(dsum, 1e-6)
    h = jnp.dot(m, w2_ref[...], preferred_element_type=jnp.float32) + b2_ref[...]
    h = jnp.where(h >= 0, h, 0.01 * h)
    rid = lax.broadcasted_iota(jnp.int32, (n_pad, d), 0)
    msk = rid < n_nodes
    hm = jnp.where(msk, h, 0.0)
    inv_n = 1.0 / n_nodes
    mu = jnp.sum(hm, axis=0, keepdims=True) * inv_n
    ex2 = jnp.sum(hm * hm, axis=0, keepdims=True) * inv_n
    var = ex2 - mu * mu
    hb = (h - mu) / jnp.sqrt(var + 1e-5) * gamma_ref[...] + beta_ref[...]
    hb = jnp.where(hb >= 0, hb, 0.01 * hb)
    h4 = jnp.dot(hb, wf1_ref[...], preferred_element_type=jnp.float32) + bf1_ref[...]
    h4 = jnp.where(h4 >= 0, h4, 0.01 * h4)
    out_ref[...] = (
        jnp.dot(h4, wf2_ref[...], preferred_element_type=jnp.float32)
        + bf2_ref[...])

  return pl.pallas_call(
      body,
      out_shape=jax.ShapeDtypeStruct((n_pad, d), jnp.float32),
  )


def kernel(x, edge_index, edge_time, node_time, edge_weight,
           W1, b1, W2, b2, gamma, beta, Wf1, bf1, Wf2, bf2):
  n_nodes, d = x.shape
  e = edge_index.shape[1]
  out_dim = Wf2.shape[1]

  n_pad = _cdiv(n_nodes, NS * CHUNK) * NS * CHUNK
  nch0, nch1 = _chunk_split(e)
  nch_max = max(nch0, nch1)
  e_pad = (nch0 + nch1) * NS * CHUNK
  pad = e_pad - e

  src = edge_index[0].astype(jnp.int32)
  dst = edge_index[1].astype(jnp.int32)
  zi = jnp.zeros((pad,), jnp.int32)
  zf = jnp.zeros((pad,), jnp.float32)

  def split_pack(a):
    c0 = a[:NS * nch0 * CHUNK].reshape(NS, nch0, CHUNK)
    c1 = a[NS * nch0 * CHUNK:].reshape(NS, nch1, CHUNK)
    z = jnp.zeros((NS, nch_max - nch0, CHUNK), a.dtype)
    z1 = jnp.zeros((NS, nch_max - nch1, CHUNK), a.dtype)
    return jnp.stack([jnp.concatenate([c0, z], axis=1),
                      jnp.concatenate([c1, z1], axis=1)])

  src4 = split_pack(jnp.concatenate([src, zi]))
  dst4 = split_pack(jnp.concatenate([dst, zi]))
  et4 = split_pack(jnp.concatenate([edge_time.astype(jnp.float32), zf]))
  ew4 = split_pack(jnp.concatenate([edge_weight.astype(jnp.float32), zf]))
  edata1 = jnp.stack([
      src4, dst4,
      lax.bitcast_convert_type(et4, jnp.int32),
      lax.bitcast_convert_type(ew4, jnp.int32),
  ], axis=3)

  nt_rows = _cdiv(n_nodes, CHUNK)
  nt2 = jnp.pad(node_time.astype(jnp.float32),
                (0, nt_rows * CHUNK - n_nodes)).reshape(nt_rows, CHUNK)
  sc1 = _make_sc1(n_nodes, nt_rows, n_pad, nch0, nch1, d)
  sc2 = _make_sc2(n_pad, nch0, nch1, d)
  tc1 = _make_tc1(n_pad, d, 1024)
  tc2 = _make_tc2(n_nodes, n_pad, d)

  agg1, deg, w4 = sc1(x, edata1, nt2)
  degt = deg.T

  h1 = tc1(agg1, degt, W1, b1.reshape(1, d))

  edata2 = jnp.stack(
      [src4, dst4, lax.bitcast_convert_type(w4, jnp.int32)], axis=3)
  agg2 = sc2(h1, edata2)

  wf1p = jnp.pad(Wf1, ((0, 0), (0, d - Wf1.shape[1])))
  bf1p = jnp.pad(bf1, (0, d - bf1.shape[0])).reshape(1, d)
  wf2p = jnp.pad(Wf2, ((0, d - Wf2.shape[0]), (0, d - Wf2.shape[1])))
  bf2p = jnp.pad(bf2, (0, d - bf2.shape[0])).reshape(1, d)

  out = tc2(agg2, degt, W2, b2.reshape(1, d),
            gamma.reshape(1, d), beta.reshape(1, d),
            wf1p, bf1p, wf2p, bf2p)
  return out[:n_nodes, :out_dim]
